# baseline (speedup 1.0000x reference)
import functools

import jax
import jax.numpy as jnp
from jax import lax
from jax.experimental import pallas as pl
from jax.experimental.pallas import tpu as pltpu
from jax.experimental.pallas import tpu_sc as plsc

N = 10000
E = 320000
C = 64
FW = 128
NSUB = 16
NCORE = 2
K = 128
NCH = E // K
NCF = NCH // NSUB
NEX = NCH - NCF * NSUB
NP = 10240
NPT = NP // NSUB

_mesh = plsc.VectorSubcoreMesh(core_axis_name="c", subcore_axis_name="s")


@functools.partial(
    pl.kernel,
    out_type=(jax.ShapeDtypeStruct((NP, FW), jnp.float32),
              jax.ShapeDtypeStruct((NP, FW), jnp.float32)),
    mesh=_mesh,
    scratch_types=[
        [pltpu.VMEM((2, K), jnp.int32)] * 4,
        pltpu.VMEM((2, K), jnp.int32),
        [pltpu.VMEM((K, FW), jnp.float32)] * 2,
        pltpu.VMEM_SHARED((NP, FW), jnp.float32),
        [pltpu.SemaphoreType.DMA] * 4,
        pltpu.SemaphoreType.DMA,
        [pltpu.SemaphoreType.DMA] * 2,
        [pltpu.SemaphoreType.DMA] * 2,
        [pltpu.SemaphoreType.DMA] * 2,
    ],
)
def _sc_segment_sum(t_hbm, e_hbm, zero_hbm, a0_hbm, a1_hbm,
                    e, et, r, acc, si, sit, sg, sg2, ss):
    c = lax.axis_index("c")
    s = lax.axis_index("s")
    cN = c * N

    pltpu.sync_copy(zero_hbm.at[pl.ds(s * NPT, NPT)],
                    acc.at[pl.ds(s * NPT, NPT)])
    plsc.subcore_barrier()

    def idx_start(j, t):
        pltpu.async_copy(e_hbm.at[s * NCF + t], e[j], si[j])

    def idx_wait(j):
        pltpu.make_async_copy(e_hbm.at[0], e[j], si[j]).wait()

    def adjust(j):
        for q in range(K // 16):
            sl = pl.ds(q * 16, 16)
            e[j][0, sl] = e[j][0, sl] + cN

    H = K // 2

    def gather_start(b, j):
        pltpu.async_copy(t_hbm.at[e[j].at[0].at[pl.ds(0, H)]],
                         r[b].at[pl.ds(0, H)], sg[b])
        pltpu.async_copy(t_hbm.at[e[j].at[0].at[pl.ds(H, H)]],
                         r[b].at[pl.ds(H, H)], sg2[b])

    def gather_wait(b, j):
        pltpu.make_async_copy(t_hbm.at[e[j].at[0].at[pl.ds(0, H)]],
                              r[b].at[pl.ds(0, H)], sg[b]).wait()
        pltpu.make_async_copy(t_hbm.at[e[j].at[0].at[pl.ds(H, H)]],
                              r[b].at[pl.ds(H, H)], sg2[b]).wait()

    def scat_start(b, j):
        pltpu.async_copy(r[b], acc.at[e[j].at[1]], ss[b], add=True)

    def scat_wait(b, j):
        pltpu.make_async_copy(r[b], acc.at[e[j].at[1]], ss[b]).wait()

    for t in range(3):
        idx_start(t, t)
    idx_wait(0)
    adjust(0)
    gather_start(0, 0)

    def quad(qi, _):
        u0 = 4 * qi
        for j in range(4):
            u = u0 + j
            b = j % 2
            nb = (j + 1) % 2

            @pl.when(u > 0)
            def _():
                scat_wait(nb, (j + 3) % 4)

            idx_start((j + 3) % 4, u + 3)
            idx_wait((j + 1) % 4)
            adjust((j + 1) % 4)
            gather_start(nb, (j + 1) % 4)
            gather_wait(b, j)
            scat_start(b, j)
        return 0

    lax.fori_loop(0, NCF // 4 - 1, quad, 0)

    has_extra = s < NEX

    scat_wait(1, 3)
    idx_start(3, NCF - 1)
    idx_wait(1)
    adjust(1)
    gather_start(1, 1)
    gather_wait(0, 0)
    scat_start(0, 0)
    scat_wait(0, 0)

    @pl.when(has_extra)
    def _():
        pltpu.async_copy(e_hbm.at[NSUB * NCF + s], et, sit)

    idx_wait(2)
    adjust(2)
    gather_start(0, 2)
    gather_wait(1, 1)
    scat_start(1, 1)
    scat_wait(1, 1)
    idx_wait(3)
    adjust(3)
    gather_start(1, 3)
    gather_wait(0, 2)
    scat_start(0, 2)
    scat_wait(0, 2)

    @pl.when(has_extra)
    def _():
        pltpu.make_async_copy(e_hbm.at[0], et, sit).wait()
        for q in range(K // 16):
            sl = pl.ds(q * 16, 16)
            et[0, sl] = et[0, sl] + cN
        pltpu.async_copy(t_hbm.at[et.at[0]], r[0], sg[0])

    gather_wait(1, 3)
    scat_start(1, 3)
    scat_wait(1, 3)

    @pl.when(has_extra)
    def _():
        pltpu.make_async_copy(t_hbm.at[et.at[0]], r[0], sg[0]).wait()
        pltpu.sync_copy(r[0], acc.at[et.at[1]], add=True)

    plsc.subcore_barrier()

    @pl.when(c == 0)
    def _():
        pltpu.sync_copy(acc.at[pl.ds(s * NPT, NPT)],
                        a0_hbm.at[pl.ds(s * NPT, NPT)])

    @pl.when(c == 1)
    def _():
        pltpu.sync_copy(acc.at[pl.ds(s * NPT, NPT)],
                        a1_hbm.at[pl.ds(s * NPT, NPT)])


_BN = 1000


def _pre_body(x_ref, y_ref, w_ref, d_ref, m_ref):
    m_ref[...] = (
        jnp.dot(x_ref[...], w_ref[0:128, :], preferred_element_type=jnp.float32)
        + jnp.dot(y_ref[...], w_ref[128:256, :],
                  preferred_element_type=jnp.float32)
        + d_ref[...])


def _pre(xf, yf, Wn, degrow):
    return pl.pallas_call(
        _pre_body,
        grid=(2, N // _BN),
        in_specs=[
            pl.BlockSpec((_BN, 128), lambda j, i: (i, 0)),
            pl.BlockSpec((_BN, 128), lambda j, i: (i, 0)),
            pl.BlockSpec((256, FW), lambda j, i: (0, j)),
            pl.BlockSpec((1, FW), lambda j, i: (0, j)),
        ],
        out_specs=pl.BlockSpec((_BN, FW), lambda j, i: (j * (N // _BN) + i, 0)),
        out_shape=jax.ShapeDtypeStruct((2 * N, FW), jnp.float32),
    )(xf, yf, Wn, degrow)


def _post_body(x_ref, y_ref, a0_ref, a1_ref, w_ref, b_ref, ox_ref, oy_ref):
    xb = x_ref[...]
    yb = y_ref[...]
    ox_ref[...] = (
        jnp.dot(xb, w_ref[0:128, 0:128], preferred_element_type=jnp.float32)
        + jnp.dot(yb, w_ref[128:256, 0:128],
                  preferred_element_type=jnp.float32)
        + a0_ref[...] + b_ref[:, 0:128])
    oy_ref[...] = (
        jnp.dot(xb, w_ref[0:128, 128:256], preferred_element_type=jnp.float32)
        + jnp.dot(yb, w_ref[128:256, 128:256],
                  preferred_element_type=jnp.float32)
        + a1_ref[...] + b_ref[:, 128:256])


def _post(xf, yf, A0, A1, Ws, bias):
    return pl.pallas_call(
        _post_body,
        grid=(N // _BN,),
        in_specs=[
            pl.BlockSpec((_BN, 128), lambda i: (i, 0)),
            pl.BlockSpec((_BN, 128), lambda i: (i, 0)),
            pl.BlockSpec((_BN, FW), lambda i: (i, 0)),
            pl.BlockSpec((_BN, FW), lambda i: (i, 0)),
            pl.BlockSpec((256, 256), lambda i: (0, 0)),
            pl.BlockSpec((1, 256), lambda i: (0, 0)),
        ],
        out_specs=[
            pl.BlockSpec((_BN, 128), lambda i: (i, 0)),
            pl.BlockSpec((_BN, 128), lambda i: (i, 0)),
        ],
        out_shape=[
            jax.ShapeDtypeStruct((N, 128), jnp.float32),
            jax.ShapeDtypeStruct((N, 128), jnp.float32),
        ],
    )(xf, yf, A0, A1, Ws, bias)


def _combine(coef_W_pairs):
    B = sum(jnp.einsum('ab,kp->akbp', cf, W) for cf, W in coef_W_pairs)
    return B.reshape(2 * C, 2 * C)


def _side(W_self, W_neigh, b_mod, pw, pb, lw, lb, hw, hb, mi):
    m0, m1, m2, m3, m4 = mi
    I2 = jnp.eye(2, dtype=jnp.float32)
    half = jnp.full((2, 2), 0.5, jnp.float32)
    alpha = [(I2, m0), (half, m1)]
    beta = [(0.5 * jnp.ones((2, 1)) * pw, m2), (lw[0], m3), (lw[1], m3 + 1),
            (hw, m4)]
    gammas = [(pb, m2), (lb[0], m3), (lb[1], m3 + 1), (hb, m4)]
    Wa_s = _combine([(cf, W_self[m]) for cf, m in alpha])
    Wb_s = _combine([(cf, W_self[m]) for cf, m in beta])
    Wa_n = _combine([(cf, W_neigh[m]) for cf, m in alpha])
    Wb_n = _combine([(cf, W_neigh[m]) for cf, m in beta])
    const = sum(g[:, None] * W_self[m].sum(axis=0)[None, :] for g, m in gammas)
    const = const + sum(b_mod[m][None, :] for m in (m0, m1, m2, m3, m3 + 1, m4))
    degc = sum(g[:, None] * W_neigh[m].sum(axis=0)[None, :] for g, m in gammas)
    return Wa_s, Wb_s, Wa_n, Wb_n, const.reshape(2 * C), degc.reshape(2 * C)


def kernel(x, y, edge_index, pool2x_w, pool2x_b, pool2y_w, pool2y_b, ls2x_w,
           ls2x_b, ls2y_w, ls2y_b, ls2x_het_w, ls2x_het_b, ls2y_het_w,
           ls2y_het_b, W_self, W_neigh, b_mod):
    Wxx_s, Wxy_s, Wxx_n, Wxy_n, const_x, degc_x = _side(
        W_self, W_neigh, b_mod, pool2x_w, pool2x_b, ls2x_w, ls2x_b,
        ls2x_het_w, ls2x_het_b, (0, 1, 2, 6, 10))
    Wyy_s, Wyx_s, Wyy_n, Wyx_n, const_y, degc_y = _side(
        W_self, W_neigh, b_mod, pool2y_w, pool2y_b, ls2y_w, ls2y_b,
        ls2y_het_w, ls2y_het_b, (3, 4, 5, 8, 11))

    Ws = jnp.block([[Wxx_s, Wyx_s], [Wxy_s, Wyy_s]])
    Wn = jnp.block([[Wxx_n, Wyx_n], [Wxy_n, Wyy_n]])
    degrow = jnp.concatenate([degc_x, degc_y]).reshape(1, 256)
    bias = jnp.concatenate([const_x, const_y]).reshape(1, 256)

    xf = x.reshape(N, 2 * C)
    yf = y.reshape(N, 2 * C)
    T = _pre(xf, yf, Wn, degrow)

    e2 = edge_index.reshape(2, NCH, K).transpose(1, 0, 2)
    zero = jnp.zeros((NP, FW), jnp.float32)
    A0, A1 = _sc_segment_sum(T, e2, zero)

    OX, OY = _post(xf, yf, A0, A1, Ws, bias)
    return OX.reshape(N, 2, C), OY.reshape(N, 2, C)


# speedup vs baseline: 1.0324x; 1.0324x over previous
import functools

import jax
import jax.numpy as jnp
from jax import lax
from jax.experimental import pallas as pl
from jax.experimental.pallas import tpu as pltpu
from jax.experimental.pallas import tpu_sc as plsc

N = 10000
E = 320000
C = 64
FW = 128
NSUB = 16
NCORE = 2
K = 128
NCH = E // K
NCF = NCH // NSUB
NEX = NCH - NCF * NSUB
NP = 10240
NPT = NP // NSUB

_mesh = plsc.VectorSubcoreMesh(core_axis_name="c", subcore_axis_name="s")


@functools.partial(
    pl.kernel,
    out_type=(jax.ShapeDtypeStruct((NP, FW), jnp.float32),
              jax.ShapeDtypeStruct((NP, FW), jnp.float32)),
    mesh=_mesh,
    scratch_types=[
        [pltpu.VMEM((2, K), jnp.int32)] * 4,
        pltpu.VMEM((2, K), jnp.int32),
        [pltpu.VMEM((K, FW), jnp.float32)] * 2,
        pltpu.VMEM_SHARED((NP, FW), jnp.float32),
        [pltpu.SemaphoreType.DMA] * 4,
        pltpu.SemaphoreType.DMA,
        [pltpu.SemaphoreType.DMA] * 2,
        [pltpu.SemaphoreType.DMA] * 2,
    ],
)
def _sc_segment_sum(t_hbm, e_hbm, zero_hbm, a0_hbm, a1_hbm,
                    e, et, r, acc, si, sit, sg, ss):
    c = lax.axis_index("c")
    s = lax.axis_index("s")
    cN = c * N

    pltpu.sync_copy(zero_hbm.at[pl.ds(s * NPT, NPT)],
                    acc.at[pl.ds(s * NPT, NPT)])
    plsc.subcore_barrier()

    def idx_start(j, t):
        pltpu.async_copy(e_hbm.at[s * NCF + t], e[j], si[j])

    def idx_wait(j):
        pltpu.make_async_copy(e_hbm.at[0], e[j], si[j]).wait()

    def adjust(j):
        for q in range(K // 16):
            sl = pl.ds(q * 16, 16)
            e[j][0, sl] = e[j][0, sl] + cN

    def gather_start(b, j):
        pltpu.async_copy(t_hbm.at[e[j].at[0]], r[b], sg[b])

    def gather_wait(b, j):
        pltpu.make_async_copy(t_hbm.at[e[j].at[0]], r[b], sg[b]).wait()

    def scat_start(b, j):
        pltpu.async_copy(r[b], acc.at[e[j].at[1]], ss[b], add=True)

    def scat_wait(b, j):
        pltpu.make_async_copy(r[b], acc.at[e[j].at[1]], ss[b]).wait()

    for t in range(3):
        idx_start(t, t)
    idx_wait(0)
    adjust(0)
    gather_start(0, 0)

    def quad(qi, _):
        u0 = 4 * qi
        for j in range(4):
            u = u0 + j
            b = j % 2
            nb = (j + 1) % 2

            @pl.when(u > 0)
            def _():
                scat_wait(nb, (j + 3) % 4)

            idx_start((j + 3) % 4, u + 3)
            idx_wait((j + 1) % 4)
            adjust((j + 1) % 4)
            gather_start(nb, (j + 1) % 4)
            gather_wait(b, j)
            scat_start(b, j)
        return 0

    lax.fori_loop(0, NCF // 4 - 1, quad, 0)

    has_extra = s < NEX

    scat_wait(1, 3)
    idx_start(3, NCF - 1)
    idx_wait(1)
    adjust(1)
    gather_start(1, 1)
    gather_wait(0, 0)
    scat_start(0, 0)
    scat_wait(0, 0)

    @pl.when(has_extra)
    def _():
        pltpu.async_copy(e_hbm.at[NSUB * NCF + s], et, sit)

    idx_wait(2)
    adjust(2)
    gather_start(0, 2)
    gather_wait(1, 1)
    scat_start(1, 1)
    scat_wait(1, 1)
    idx_wait(3)
    adjust(3)
    gather_start(1, 3)
    gather_wait(0, 2)
    scat_start(0, 2)
    scat_wait(0, 2)

    @pl.when(has_extra)
    def _():
        pltpu.make_async_copy(e_hbm.at[0], et, sit).wait()
        for q in range(K // 16):
            sl = pl.ds(q * 16, 16)
            et[0, sl] = et[0, sl] + cN
        pltpu.async_copy(t_hbm.at[et.at[0]], r[0], sg[0])

    gather_wait(1, 3)
    scat_start(1, 3)
    scat_wait(1, 3)

    @pl.when(has_extra)
    def _():
        pltpu.make_async_copy(t_hbm.at[et.at[0]], r[0], sg[0]).wait()
        pltpu.sync_copy(r[0], acc.at[et.at[1]], add=True)

    plsc.subcore_barrier()

    @pl.when(c == 0)
    def _():
        pltpu.sync_copy(acc.at[pl.ds(s * NPT, NPT)],
                        a0_hbm.at[pl.ds(s * NPT, NPT)])

    @pl.when(c == 1)
    def _():
        pltpu.sync_copy(acc.at[pl.ds(s * NPT, NPT)],
                        a1_hbm.at[pl.ds(s * NPT, NPT)])


_BN = 1000


def _pre_body(x_ref, y_ref, w_ref, d_ref, m_ref):
    m_ref[...] = (
        jnp.dot(x_ref[...], w_ref[0:128, :], preferred_element_type=jnp.float32)
        + jnp.dot(y_ref[...], w_ref[128:256, :],
                  preferred_element_type=jnp.float32)
        + d_ref[...])


def _pre(xf, yf, Wn, degrow):
    return pl.pallas_call(
        _pre_body,
        grid=(2, N // _BN),
        in_specs=[
            pl.BlockSpec((_BN, 128), lambda j, i: (i, 0)),
            pl.BlockSpec((_BN, 128), lambda j, i: (i, 0)),
            pl.BlockSpec((256, FW), lambda j, i: (0, j)),
            pl.BlockSpec((1, FW), lambda j, i: (0, j)),
        ],
        out_specs=pl.BlockSpec((_BN, FW), lambda j, i: (j * (N // _BN) + i, 0)),
        out_shape=jax.ShapeDtypeStruct((2 * N, FW), jnp.float32),
    )(xf, yf, Wn, degrow)


def _post_body(x_ref, y_ref, a0_ref, a1_ref, w_ref, b_ref, ox_ref, oy_ref):
    xb = x_ref[...]
    yb = y_ref[...]
    ox_ref[...] = (
        jnp.dot(xb, w_ref[0:128, 0:128], preferred_element_type=jnp.float32)
        + jnp.dot(yb, w_ref[128:256, 0:128],
                  preferred_element_type=jnp.float32)
        + a0_ref[...] + b_ref[:, 0:128])
    oy_ref[...] = (
        jnp.dot(xb, w_ref[0:128, 128:256], preferred_element_type=jnp.float32)
        + jnp.dot(yb, w_ref[128:256, 128:256],
                  preferred_element_type=jnp.float32)
        + a1_ref[...] + b_ref[:, 128:256])


def _post(xf, yf, A0, A1, Ws, bias):
    return pl.pallas_call(
        _post_body,
        grid=(N // _BN,),
        in_specs=[
            pl.BlockSpec((_BN, 128), lambda i: (i, 0)),
            pl.BlockSpec((_BN, 128), lambda i: (i, 0)),
            pl.BlockSpec((_BN, FW), lambda i: (i, 0)),
            pl.BlockSpec((_BN, FW), lambda i: (i, 0)),
            pl.BlockSpec((256, 256), lambda i: (0, 0)),
            pl.BlockSpec((1, 256), lambda i: (0, 0)),
        ],
        out_specs=[
            pl.BlockSpec((_BN, 128), lambda i: (i, 0)),
            pl.BlockSpec((_BN, 128), lambda i: (i, 0)),
        ],
        out_shape=[
            jax.ShapeDtypeStruct((N, 128), jnp.float32),
            jax.ShapeDtypeStruct((N, 128), jnp.float32),
        ],
    )(xf, yf, A0, A1, Ws, bias)


def _combine(coef_W_pairs):
    B = sum(jnp.einsum('ab,kp->akbp', cf, W) for cf, W in coef_W_pairs)
    return B.reshape(2 * C, 2 * C)


def _side(W_self, W_neigh, b_mod, pw, pb, lw, lb, hw, hb, mi):
    m0, m1, m2, m3, m4 = mi
    I2 = jnp.eye(2, dtype=jnp.float32)
    half = jnp.full((2, 2), 0.5, jnp.float32)
    alpha = [(I2, m0), (half, m1)]
    beta = [(0.5 * jnp.ones((2, 1)) * pw, m2), (lw[0], m3), (lw[1], m3 + 1),
            (hw, m4)]
    gammas = [(pb, m2), (lb[0], m3), (lb[1], m3 + 1), (hb, m4)]
    Wa_s = _combine([(cf, W_self[m]) for cf, m in alpha])
    Wb_s = _combine([(cf, W_self[m]) for cf, m in beta])
    Wa_n = _combine([(cf, W_neigh[m]) for cf, m in alpha])
    Wb_n = _combine([(cf, W_neigh[m]) for cf, m in beta])
    const = sum(g[:, None] * W_self[m].sum(axis=0)[None, :] for g, m in gammas)
    const = const + sum(b_mod[m][None, :] for m in (m0, m1, m2, m3, m3 + 1, m4))
    degc = sum(g[:, None] * W_neigh[m].sum(axis=0)[None, :] for g, m in gammas)
    return Wa_s, Wb_s, Wa_n, Wb_n, const.reshape(2 * C), degc.reshape(2 * C)


def kernel(x, y, edge_index, pool2x_w, pool2x_b, pool2y_w, pool2y_b, ls2x_w,
           ls2x_b, ls2y_w, ls2y_b, ls2x_het_w, ls2x_het_b, ls2y_het_w,
           ls2y_het_b, W_self, W_neigh, b_mod):
    Wxx_s, Wxy_s, Wxx_n, Wxy_n, const_x, degc_x = _side(
        W_self, W_neigh, b_mod, pool2x_w, pool2x_b, ls2x_w, ls2x_b,
        ls2x_het_w, ls2x_het_b, (0, 1, 2, 6, 10))
    Wyy_s, Wyx_s, Wyy_n, Wyx_n, const_y, degc_y = _side(
        W_self, W_neigh, b_mod, pool2y_w, pool2y_b, ls2y_w, ls2y_b,
        ls2y_het_w, ls2y_het_b, (3, 4, 5, 8, 11))

    Ws = jnp.block([[Wxx_s, Wyx_s], [Wxy_s, Wyy_s]])
    Wn = jnp.block([[Wxx_n, Wyx_n], [Wxy_n, Wyy_n]])
    degrow = jnp.concatenate([degc_x, degc_y]).reshape(1, 256)
    bias = jnp.concatenate([const_x, const_y]).reshape(1, 256)

    xf = x.reshape(N, 2 * C)
    yf = y.reshape(N, 2 * C)
    T = _pre(xf, yf, Wn, degrow)

    e2 = edge_index.reshape(2, NCH, K).transpose(1, 0, 2)
    zero = jnp.zeros((NP, FW), jnp.float32)
    A0, A1 = _sc_segment_sum(T, e2, zero)

    OX, OY = _post(xf, yf, A0, A1, Ws, bias)
    return OX.reshape(N, 2, C), OY.reshape(N, 2, C)


# speedup vs baseline: 1.0559x; 1.0227x over previous
import functools

import jax
import jax.numpy as jnp
from jax import lax
from jax.experimental import pallas as pl
from jax.experimental.pallas import tpu as pltpu
from jax.experimental.pallas import tpu_sc as plsc

N = 10000
E = 320000
C = 64
FW = 128
NSUB = 16
NCORE = 2
K = 128
NCH = E // K
NCF = NCH // NSUB
NEX = NCH - NCF * NSUB
NP = 10240
NPT = NP // NSUB

_mesh = plsc.VectorSubcoreMesh(core_axis_name="c", subcore_axis_name="s")


@functools.partial(
    pl.kernel,
    out_type=(jax.ShapeDtypeStruct((NP, FW), jnp.float32),
              jax.ShapeDtypeStruct((NP, FW), jnp.float32)),
    mesh=_mesh,
    scratch_types=[
        [pltpu.VMEM((2, K), jnp.int32)] * 4,
        pltpu.VMEM((2, K), jnp.int32),
        [pltpu.VMEM((K, FW), jnp.float32)] * 2,
        pltpu.VMEM_SHARED((NP, FW), jnp.float32),
        [pltpu.SemaphoreType.DMA] * 4,
        pltpu.SemaphoreType.DMA,
        [pltpu.SemaphoreType.DMA] * 2,
        [pltpu.SemaphoreType.DMA] * 2,
    ],
)
def _sc_segment_sum(t_hbm, e_hbm, zero_hbm, a0_hbm, a1_hbm,
                    e, et, r, acc, si, sit, sg, ss):
    c = lax.axis_index("c")
    s = lax.axis_index("s")
    cN = c * N

    pltpu.sync_copy(zero_hbm.at[pl.ds(s * NPT, NPT)],
                    acc.at[pl.ds(s * NPT, NPT)])
    plsc.subcore_barrier()

    def idx_start(j, t):
        pltpu.async_copy(e_hbm.at[s * NCF + t], e[j], si[j])

    def idx_wait(j):
        pltpu.make_async_copy(e_hbm.at[0], e[j], si[j]).wait()

    def adjust(j):
        for q in range(K // 16):
            sl = pl.ds(q * 16, 16)
            e[j][0, sl] = e[j][0, sl] + cN

    def gather_start(b, j):
        pltpu.async_copy(t_hbm.at[e[j].at[0]], r[b], sg[b])

    def gather_wait(b, j):
        pltpu.make_async_copy(t_hbm.at[e[j].at[0]], r[b], sg[b]).wait()

    def scat_start(b, j):
        pltpu.async_copy(r[b], acc.at[e[j].at[1]], ss[b], add=True)

    def scat_wait(b, j):
        pltpu.make_async_copy(r[b], acc.at[e[j].at[1]], ss[b]).wait()

    for t in range(3):
        idx_start(t, t)
    idx_wait(0)
    adjust(0)
    gather_start(0, 0)

    def quad(qi, _):
        u0 = 4 * qi
        for j in range(4):
            u = u0 + j
            b = j % 2
            nb = (j + 1) % 2

            @pl.when(u > 0)
            def _():
                scat_wait(nb, (j + 3) % 4)

            idx_start((j + 3) % 4, u + 3)
            idx_wait((j + 1) % 4)
            adjust((j + 1) % 4)
            gather_start(nb, (j + 1) % 4)
            gather_wait(b, j)
            scat_start(b, j)
        return 0

    lax.fori_loop(0, NCF // 4 - 1, quad, 0)

    has_extra = s < NEX

    scat_wait(1, 3)
    idx_start(3, NCF - 1)
    idx_wait(1)
    adjust(1)
    gather_start(1, 1)
    gather_wait(0, 0)
    scat_start(0, 0)
    scat_wait(0, 0)

    @pl.when(has_extra)
    def _():
        pltpu.async_copy(e_hbm.at[NSUB * NCF + s], et, sit)

    idx_wait(2)
    adjust(2)
    gather_start(0, 2)
    gather_wait(1, 1)
    scat_start(1, 1)
    scat_wait(1, 1)
    idx_wait(3)
    adjust(3)
    gather_start(1, 3)
    gather_wait(0, 2)
    scat_start(0, 2)
    scat_wait(0, 2)

    @pl.when(has_extra)
    def _():
        pltpu.make_async_copy(e_hbm.at[0], et, sit).wait()
        for q in range(K // 16):
            sl = pl.ds(q * 16, 16)
            et[0, sl] = et[0, sl] + cN
        pltpu.async_copy(t_hbm.at[et.at[0]], r[0], sg[0])

    gather_wait(1, 3)
    scat_start(1, 3)
    scat_wait(1, 3)

    @pl.when(has_extra)
    def _():
        pltpu.make_async_copy(t_hbm.at[et.at[0]], r[0], sg[0]).wait()
        pltpu.sync_copy(r[0], acc.at[et.at[1]], add=True)

    plsc.subcore_barrier()

    @pl.when(c == 0)
    def _():
        pltpu.sync_copy(acc.at[pl.ds(s * NPT, NPT)],
                        a0_hbm.at[pl.ds(s * NPT, NPT)])

    @pl.when(c == 1)
    def _():
        pltpu.sync_copy(acc.at[pl.ds(s * NPT, NPT)],
                        a1_hbm.at[pl.ds(s * NPT, NPT)])


_BN = 2000


def _pre_body(x_ref, y_ref, w_ref, d_ref, m_ref):
    m_ref[...] = (
        jnp.dot(x_ref[...], w_ref[0:128, :], preferred_element_type=jnp.float32)
        + jnp.dot(y_ref[...], w_ref[128:256, :],
                  preferred_element_type=jnp.float32)
        + d_ref[...])


def _pre(xf, yf, Wn, degrow):
    return pl.pallas_call(
        _pre_body,
        grid=(2, N // _BN),
        in_specs=[
            pl.BlockSpec((_BN, 128), lambda j, i: (i, 0)),
            pl.BlockSpec((_BN, 128), lambda j, i: (i, 0)),
            pl.BlockSpec((256, FW), lambda j, i: (0, j)),
            pl.BlockSpec((1, FW), lambda j, i: (0, j)),
        ],
        out_specs=pl.BlockSpec((_BN, FW), lambda j, i: (j * (N // _BN) + i, 0)),
        out_shape=jax.ShapeDtypeStruct((2 * N, FW), jnp.float32),
    )(xf, yf, Wn, degrow)


def _post_body(x_ref, y_ref, a0_ref, a1_ref, w_ref, b_ref, ox_ref, oy_ref):
    xb = x_ref[...]
    yb = y_ref[...]
    ox_ref[...] = (
        jnp.dot(xb, w_ref[0:128, 0:128], preferred_element_type=jnp.float32)
        + jnp.dot(yb, w_ref[128:256, 0:128],
                  preferred_element_type=jnp.float32)
        + a0_ref[...] + b_ref[:, 0:128])
    oy_ref[...] = (
        jnp.dot(xb, w_ref[0:128, 128:256], preferred_element_type=jnp.float32)
        + jnp.dot(yb, w_ref[128:256, 128:256],
                  preferred_element_type=jnp.float32)
        + a1_ref[...] + b_ref[:, 128:256])


def _post(xf, yf, A0, A1, Ws, bias):
    return pl.pallas_call(
        _post_body,
        grid=(N // _BN,),
        in_specs=[
            pl.BlockSpec((_BN, 128), lambda i: (i, 0)),
            pl.BlockSpec((_BN, 128), lambda i: (i, 0)),
            pl.BlockSpec((_BN, FW), lambda i: (i, 0)),
            pl.BlockSpec((_BN, FW), lambda i: (i, 0)),
            pl.BlockSpec((256, 256), lambda i: (0, 0)),
            pl.BlockSpec((1, 256), lambda i: (0, 0)),
        ],
        out_specs=[
            pl.BlockSpec((_BN, 128), lambda i: (i, 0)),
            pl.BlockSpec((_BN, 128), lambda i: (i, 0)),
        ],
        out_shape=[
            jax.ShapeDtypeStruct((N, 128), jnp.float32),
            jax.ShapeDtypeStruct((N, 128), jnp.float32),
        ],
    )(xf, yf, A0, A1, Ws, bias)


def _combine(coef_W_pairs):
    B = sum(jnp.einsum('ab,kp->akbp', cf, W) for cf, W in coef_W_pairs)
    return B.reshape(2 * C, 2 * C)


def _side(W_self, W_neigh, b_mod, pw, pb, lw, lb, hw, hb, mi):
    m0, m1, m2, m3, m4 = mi
    I2 = jnp.eye(2, dtype=jnp.float32)
    half = jnp.full((2, 2), 0.5, jnp.float32)
    alpha = [(I2, m0), (half, m1)]
    beta = [(0.5 * jnp.ones((2, 1)) * pw, m2), (lw[0], m3), (lw[1], m3 + 1),
            (hw, m4)]
    gammas = [(pb, m2), (lb[0], m3), (lb[1], m3 + 1), (hb, m4)]
    Wa_s = _combine([(cf, W_self[m]) for cf, m in alpha])
    Wb_s = _combine([(cf, W_self[m]) for cf, m in beta])
    Wa_n = _combine([(cf, W_neigh[m]) for cf, m in alpha])
    Wb_n = _combine([(cf, W_neigh[m]) for cf, m in beta])
    const = sum(g[:, None] * W_self[m].sum(axis=0)[None, :] for g, m in gammas)
    const = const + sum(b_mod[m][None, :] for m in (m0, m1, m2, m3, m3 + 1, m4))
    degc = sum(g[:, None] * W_neigh[m].sum(axis=0)[None, :] for g, m in gammas)
    return Wa_s, Wb_s, Wa_n, Wb_n, const.reshape(2 * C), degc.reshape(2 * C)


def kernel(x, y, edge_index, pool2x_w, pool2x_b, pool2y_w, pool2y_b, ls2x_w,
           ls2x_b, ls2y_w, ls2y_b, ls2x_het_w, ls2x_het_b, ls2y_het_w,
           ls2y_het_b, W_self, W_neigh, b_mod):
    Wxx_s, Wxy_s, Wxx_n, Wxy_n, const_x, degc_x = _side(
        W_self, W_neigh, b_mod, pool2x_w, pool2x_b, ls2x_w, ls2x_b,
        ls2x_het_w, ls2x_het_b, (0, 1, 2, 6, 10))
    Wyy_s, Wyx_s, Wyy_n, Wyx_n, const_y, degc_y = _side(
        W_self, W_neigh, b_mod, pool2y_w, pool2y_b, ls2y_w, ls2y_b,
        ls2y_het_w, ls2y_het_b, (3, 4, 5, 8, 11))

    Ws = jnp.block([[Wxx_s, Wyx_s], [Wxy_s, Wyy_s]])
    Wn = jnp.block([[Wxx_n, Wyx_n], [Wxy_n, Wyy_n]])
    degrow = jnp.concatenate([degc_x, degc_y]).reshape(1, 256)
    bias = jnp.concatenate([const_x, const_y]).reshape(1, 256)

    xf = x.reshape(N, 2 * C)
    yf = y.reshape(N, 2 * C)
    T = _pre(xf, yf, Wn, degrow)

    e2 = edge_index.reshape(2, NCH, K).transpose(1, 0, 2)
    zero = jnp.zeros((NP, FW), jnp.float32)
    A0, A1 = _sc_segment_sum(T, e2, zero)

    OX, OY = _post(xf, yf, A0, A1, Ws, bias)
    return OX.reshape(N, 2, C), OY.reshape(N, 2, C)
